# Initial kernel scaffold; baseline (speedup 1.0000x reference)
import jax
import jax.numpy as jnp
from jax.experimental import pallas as pl


def kernel(x, rbf, idx_i, num_nodes, W_rbf, W1, b1, W2, b2, W3, b3, W_out):
    raise NotImplementedError("write your pallas kernel here")


# speedup vs baseline: 2.3444x; 2.3444x over previous
import functools

import jax
import jax.numpy as jnp
from jax import lax
from jax.experimental import pallas as pl
from jax.experimental.pallas import tpu as pltpu
from jax.experimental.pallas import tpu_sc as plsc

N_NODES_STATIC = 10000
N_EDGES = 320000
EDGE_DIM = 128
N_RADIAL = 16

NC = 2
NS = 16
NW = NC * NS

IDX_ROW = 128
GROUP_ROWS = 2
GROUP_EDGES = GROUP_ROWS * IDX_ROW
N_GROUPS = N_EDGES // GROUP_EDGES
ZROWS = 624
ZTAIL = N_NODES_STATIC - NS * ZROWS


def _gate_body(rbf_ref, x_ref, w_ref, h_ref):
    g = jnp.dot(rbf_ref[...], w_ref[...], preferred_element_type=jnp.float32)
    h_ref[...] = g * x_ref[...]


def _gating(rbf, x, W_rbf):
    BE = 2000
    grid = (N_EDGES // BE,)
    return pl.pallas_call(
        _gate_body,
        grid=grid,
        in_specs=[
            pl.BlockSpec((BE, N_RADIAL), lambda i: (i, 0)),
            pl.BlockSpec((BE, EDGE_DIM), lambda i: (i, 0)),
            pl.BlockSpec((N_RADIAL, EDGE_DIM), lambda i: (0, 0)),
        ],
        out_specs=pl.BlockSpec((BE, EDGE_DIM), lambda i: (i, 0)),
        out_shape=jax.ShapeDtypeStruct((N_EDGES, EDGE_DIM), jnp.float32),
    )(rbf, x, W_rbf)


def _sc_scatter_body(h_hbm, idx_hbm, zeros_hbm, out_hbm, idx_v, rows_v, acc):
    c = lax.axis_index("c")
    s = lax.axis_index("s")
    w = s * NC + c

    pltpu.sync_copy(
        zeros_hbm.at[pl.ds(s * ZROWS, ZROWS)],
        acc.at[pl.ds(s * ZROWS, ZROWS)],
    )

    @pl.when(s == NS - 1)
    def _zero_tail():
        pltpu.sync_copy(
            zeros_hbm.at[pl.ds(NS * ZROWS, ZTAIL)],
            acc.at[pl.ds(NS * ZROWS, ZTAIL)],
        )

    plsc.subcore_barrier()

    def group_body(i, carry):
        g = w + NW * i
        e0 = g * GROUP_EDGES
        pltpu.sync_copy(h_hbm.at[pl.ds(e0, GROUP_EDGES)], rows_v)
        for j in range(GROUP_ROWS):
            pltpu.sync_copy(
                idx_hbm.at[pl.ds(e0 + j * IDX_ROW, IDX_ROW)], idx_v.at[j])
        for j in range(GROUP_ROWS):
            pltpu.sync_copy(
                rows_v.at[pl.ds(j * IDX_ROW, IDX_ROW)],
                acc.at[idx_v.at[j]],
                add=True,
            )
        return carry

    ng = (N_GROUPS - w + NW - 1) // NW
    lax.fori_loop(0, ng, group_body, 0)

    plsc.subcore_barrier()
    pltpu.sync_copy(
        acc.at[pl.ds(s * ZROWS, ZROWS)],
        out_hbm.at[c, pl.ds(s * ZROWS, ZROWS)],
    )

    @pl.when(s == NS - 1)
    def _write_tail():
        pltpu.sync_copy(
            acc.at[pl.ds(NS * ZROWS, ZTAIL)],
            out_hbm.at[c, pl.ds(NS * ZROWS, ZTAIL)],
        )


def _sc_scatter(h, idx1d, zeros):
    mesh = plsc.VectorSubcoreMesh(core_axis_name="c", subcore_axis_name="s")
    kfn = pl.kernel(
        _sc_scatter_body,
        out_type=jax.ShapeDtypeStruct((NC, N_NODES_STATIC, EDGE_DIM), jnp.float32),
        mesh=mesh,
        scratch_types=[
            pltpu.VMEM((GROUP_ROWS, IDX_ROW), jnp.int32),
            pltpu.VMEM((GROUP_EDGES, EDGE_DIM), jnp.float32),
            pltpu.VMEM_SHARED((N_NODES_STATIC, EDGE_DIM), jnp.float32),
        ],
    )
    return kfn(h, idx1d, zeros)


def _mlp_body(p_ref, w1, b1, w2, b2, w3, b3, wo, o_ref):
    a = p_ref[0] + p_ref[1]
    for wref, bref in ((w1, b1), (w2, b2), (w3, b3)):
        a = jnp.dot(a, wref[...], preferred_element_type=jnp.float32) + bref[...]
        a = a * (1.0 / (1.0 + jnp.exp(-a)))
    o_ref[...] = jnp.dot(a, wo[...], preferred_element_type=jnp.float32)


def _mlp(partials, W1, b1, W2, b2, W3, b3, W_out):
    BN = 2000
    grid = (N_NODES_STATIC // BN,)
    full = lambda shape: pl.BlockSpec(shape, lambda i: tuple(0 for _ in shape))
    return pl.pallas_call(
        _mlp_body,
        grid=grid,
        in_specs=[
            pl.BlockSpec((NC, BN, EDGE_DIM), lambda i: (0, i, 0)),
            full((EDGE_DIM, EDGE_DIM)),
            full((1, EDGE_DIM)),
            full((EDGE_DIM, EDGE_DIM)),
            full((1, EDGE_DIM)),
            full((EDGE_DIM, EDGE_DIM)),
            full((1, EDGE_DIM)),
            full((EDGE_DIM, 1)),
        ],
        out_specs=pl.BlockSpec((BN, 1), lambda i: (i, 0)),
        out_shape=jax.ShapeDtypeStruct((N_NODES_STATIC, 1), jnp.float32),
    )(partials, W1, b1.reshape(1, -1), W2, b2.reshape(1, -1),
      W3, b3.reshape(1, -1), W_out)


def kernel(x, rbf, idx_i, num_nodes, W_rbf, W1, b1, W2, b2, W3, b3, W_out):
    idx = idx_i.astype(jnp.int32) + (
        jnp.asarray(num_nodes, jnp.int32) - N_NODES_STATIC)
    zeros = jnp.zeros((N_NODES_STATIC, EDGE_DIM), jnp.float32)

    h = _gating(rbf, x, W_rbf)
    partials = _sc_scatter(h, idx, zeros)
    return _mlp(partials, W1, b1, W2, b2, W3, b3, W_out)


# speedup vs baseline: 2.5840x; 1.1022x over previous
import functools

import jax
import jax.numpy as jnp
from jax import lax
from jax.experimental import pallas as pl
from jax.experimental.pallas import tpu as pltpu
from jax.experimental.pallas import tpu_sc as plsc

N_NODES_STATIC = 10000
N_EDGES = 320000
EDGE_DIM = 128
N_RADIAL = 16

NC = 2
NS = 16
NW = NC * NS

IDX_ROW = 128
GROUP_ROWS = 2
GROUP_EDGES = GROUP_ROWS * IDX_ROW
N_GROUPS = N_EDGES // GROUP_EDGES
ZROWS = 624
ZTAIL = N_NODES_STATIC - NS * ZROWS


def _gate_body(rbf_ref, x_ref, w_ref, h_ref):
    g = jnp.dot(rbf_ref[...], w_ref[...], preferred_element_type=jnp.float32)
    h_ref[...] = g * x_ref[...]


def _gating(rbf, x, W_rbf):
    BE = 8000
    grid = (N_EDGES // BE,)
    return pl.pallas_call(
        _gate_body,
        grid=grid,
        in_specs=[
            pl.BlockSpec((BE, N_RADIAL), lambda i: (i, 0)),
            pl.BlockSpec((BE, EDGE_DIM), lambda i: (i, 0)),
            pl.BlockSpec((N_RADIAL, EDGE_DIM), lambda i: (0, 0)),
        ],
        out_specs=pl.BlockSpec((BE, EDGE_DIM), lambda i: (i, 0)),
        out_shape=jax.ShapeDtypeStruct((N_EDGES, EDGE_DIM), jnp.float32),
    )(rbf, x, W_rbf)


def _sc_scatter_body(h_hbm, idx_hbm, zeros_hbm, out_hbm, idx_v, rows_v, acc):
    c = lax.axis_index("c")
    s = lax.axis_index("s")
    w = s * NC + c

    pltpu.sync_copy(
        zeros_hbm.at[pl.ds(s * ZROWS, ZROWS)],
        acc.at[pl.ds(s * ZROWS, ZROWS)],
    )

    @pl.when(s == NS - 1)
    def _zero_tail():
        pltpu.sync_copy(
            zeros_hbm.at[pl.ds(NS * ZROWS, ZTAIL)],
            acc.at[pl.ds(NS * ZROWS, ZTAIL)],
        )

    plsc.subcore_barrier()

    def group_body(i, carry):
        g = w + NW * i
        e0 = g * GROUP_EDGES
        pltpu.sync_copy(h_hbm.at[pl.ds(e0, GROUP_EDGES)], rows_v)
        for j in range(GROUP_ROWS):
            pltpu.sync_copy(
                idx_hbm.at[pl.ds(e0 + j * IDX_ROW, IDX_ROW)], idx_v.at[j])
        for j in range(GROUP_ROWS):
            pltpu.sync_copy(
                rows_v.at[pl.ds(j * IDX_ROW, IDX_ROW)],
                acc.at[idx_v.at[j]],
                add=True,
            )
        return carry

    ng = (N_GROUPS - w + NW - 1) // NW
    lax.fori_loop(0, ng, group_body, 0)

    plsc.subcore_barrier()
    pltpu.sync_copy(
        acc.at[pl.ds(s * ZROWS, ZROWS)],
        out_hbm.at[c, pl.ds(s * ZROWS, ZROWS)],
    )

    @pl.when(s == NS - 1)
    def _write_tail():
        pltpu.sync_copy(
            acc.at[pl.ds(NS * ZROWS, ZTAIL)],
            out_hbm.at[c, pl.ds(NS * ZROWS, ZTAIL)],
        )


def _sc_scatter(h, idx1d, zeros):
    mesh = plsc.VectorSubcoreMesh(core_axis_name="c", subcore_axis_name="s")
    kfn = pl.kernel(
        _sc_scatter_body,
        out_type=jax.ShapeDtypeStruct((NC, N_NODES_STATIC, EDGE_DIM), jnp.float32),
        mesh=mesh,
        scratch_types=[
            pltpu.VMEM((GROUP_ROWS, IDX_ROW), jnp.int32),
            pltpu.VMEM((GROUP_EDGES, EDGE_DIM), jnp.float32),
            pltpu.VMEM_SHARED((N_NODES_STATIC, EDGE_DIM), jnp.float32),
        ],
    )
    return kfn(h, idx1d, zeros)


def _mlp_body(p_ref, w1, b1, w2, b2, w3, b3, wo, o_ref):
    a = p_ref[0] + p_ref[1]
    for wref, bref in ((w1, b1), (w2, b2), (w3, b3)):
        a = jnp.dot(a, wref[...], preferred_element_type=jnp.float32) + bref[...]
        a = a * (1.0 / (1.0 + jnp.exp(-a)))
    o_ref[...] = jnp.dot(a, wo[...], preferred_element_type=jnp.float32)


def _mlp(partials, W1, b1, W2, b2, W3, b3, W_out):
    BN = 2000
    grid = (N_NODES_STATIC // BN,)
    full = lambda shape: pl.BlockSpec(shape, lambda i: tuple(0 for _ in shape))
    return pl.pallas_call(
        _mlp_body,
        grid=grid,
        in_specs=[
            pl.BlockSpec((NC, BN, EDGE_DIM), lambda i: (0, i, 0)),
            full((EDGE_DIM, EDGE_DIM)),
            full((1, EDGE_DIM)),
            full((EDGE_DIM, EDGE_DIM)),
            full((1, EDGE_DIM)),
            full((EDGE_DIM, EDGE_DIM)),
            full((1, EDGE_DIM)),
            full((EDGE_DIM, 1)),
        ],
        out_specs=pl.BlockSpec((BN, 1), lambda i: (i, 0)),
        out_shape=jax.ShapeDtypeStruct((N_NODES_STATIC, 1), jnp.float32),
    )(partials, W1, b1.reshape(1, -1), W2, b2.reshape(1, -1),
      W3, b3.reshape(1, -1), W_out)


def kernel(x, rbf, idx_i, num_nodes, W_rbf, W1, b1, W2, b2, W3, b3, W_out):
    idx = idx_i.astype(jnp.int32) + (
        jnp.asarray(num_nodes, jnp.int32) - N_NODES_STATIC)
    zeros = jnp.zeros((N_NODES_STATIC, EDGE_DIM), jnp.float32)

    h = _gating(rbf, x, W_rbf)
    partials = _sc_scatter(h, idx, zeros)
    return _mlp(partials, W1, b1, W2, b2, W3, b3, W_out)


# speedup vs baseline: 2.5873x; 1.0013x over previous
import functools

import jax
import jax.numpy as jnp
from jax import lax
from jax.experimental import pallas as pl
from jax.experimental.pallas import tpu as pltpu
from jax.experimental.pallas import tpu_sc as plsc

N_NODES_STATIC = 10000
N_EDGES = 320000
EDGE_DIM = 128
N_RADIAL = 16

NC = 2
NS = 16
NW = NC * NS

IDX_ROW = 128
GROUP_ROWS = 2
GROUP_EDGES = GROUP_ROWS * IDX_ROW
N_GROUPS = N_EDGES // GROUP_EDGES
ZROWS = 624
ZTAIL = N_NODES_STATIC - NS * ZROWS


def _gate_body(rbf_ref, x_ref, w_ref, h_ref):
    g = jnp.dot(rbf_ref[...], w_ref[...], preferred_element_type=jnp.float32)
    h_ref[...] = g * x_ref[...]


def _gating(rbf, x, W_rbf):
    BE = 16000
    grid = (N_EDGES // BE,)
    return pl.pallas_call(
        _gate_body,
        grid=grid,
        in_specs=[
            pl.BlockSpec((BE, N_RADIAL), lambda i: (i, 0)),
            pl.BlockSpec((BE, EDGE_DIM), lambda i: (i, 0)),
            pl.BlockSpec((N_RADIAL, EDGE_DIM), lambda i: (0, 0)),
        ],
        out_specs=pl.BlockSpec((BE, EDGE_DIM), lambda i: (i, 0)),
        out_shape=jax.ShapeDtypeStruct((N_EDGES, EDGE_DIM), jnp.float32),
    )(rbf, x, W_rbf)


def _sc_scatter_body(h_hbm, idx_hbm, zeros_hbm, out_hbm, idx_v, rows_v, acc):
    c = lax.axis_index("c")
    s = lax.axis_index("s")
    w = s * NC + c

    pltpu.sync_copy(
        zeros_hbm.at[pl.ds(s * ZROWS, ZROWS)],
        acc.at[pl.ds(s * ZROWS, ZROWS)],
    )

    @pl.when(s == NS - 1)
    def _zero_tail():
        pltpu.sync_copy(
            zeros_hbm.at[pl.ds(NS * ZROWS, ZTAIL)],
            acc.at[pl.ds(NS * ZROWS, ZTAIL)],
        )

    plsc.subcore_barrier()

    def group_body(i, carry):
        g = w + NW * i
        e0 = g * GROUP_EDGES
        pltpu.sync_copy(h_hbm.at[pl.ds(e0, GROUP_EDGES)], rows_v)
        for j in range(GROUP_ROWS):
            pltpu.sync_copy(
                idx_hbm.at[pl.ds(e0 + j * IDX_ROW, IDX_ROW)], idx_v.at[j])
        for j in range(GROUP_ROWS):
            pltpu.sync_copy(
                rows_v.at[pl.ds(j * IDX_ROW, IDX_ROW)],
                acc.at[idx_v.at[j]],
                add=True,
            )
        return carry

    ng = (N_GROUPS - w + NW - 1) // NW
    lax.fori_loop(0, ng, group_body, 0)

    plsc.subcore_barrier()
    pltpu.sync_copy(
        acc.at[pl.ds(s * ZROWS, ZROWS)],
        out_hbm.at[c, pl.ds(s * ZROWS, ZROWS)],
    )

    @pl.when(s == NS - 1)
    def _write_tail():
        pltpu.sync_copy(
            acc.at[pl.ds(NS * ZROWS, ZTAIL)],
            out_hbm.at[c, pl.ds(NS * ZROWS, ZTAIL)],
        )


def _sc_scatter(h, idx1d, zeros):
    mesh = plsc.VectorSubcoreMesh(core_axis_name="c", subcore_axis_name="s")
    kfn = pl.kernel(
        _sc_scatter_body,
        out_type=jax.ShapeDtypeStruct((NC, N_NODES_STATIC, EDGE_DIM), jnp.float32),
        mesh=mesh,
        scratch_types=[
            pltpu.VMEM((GROUP_ROWS, IDX_ROW), jnp.int32),
            pltpu.VMEM((GROUP_EDGES, EDGE_DIM), jnp.float32),
            pltpu.VMEM_SHARED((N_NODES_STATIC, EDGE_DIM), jnp.float32),
        ],
    )
    return kfn(h, idx1d, zeros)


def _mlp_body(p_ref, w1, b1, w2, b2, w3, b3, wo, o_ref):
    a = p_ref[0] + p_ref[1]
    for wref, bref in ((w1, b1), (w2, b2), (w3, b3)):
        a = jnp.dot(a, wref[...], preferred_element_type=jnp.float32) + bref[...]
        a = a * (1.0 / (1.0 + jnp.exp(-a)))
    o_ref[...] = jnp.dot(a, wo[...], preferred_element_type=jnp.float32)


def _mlp(partials, W1, b1, W2, b2, W3, b3, W_out):
    BN = 2000
    grid = (N_NODES_STATIC // BN,)
    full = lambda shape: pl.BlockSpec(shape, lambda i: tuple(0 for _ in shape))
    return pl.pallas_call(
        _mlp_body,
        grid=grid,
        in_specs=[
            pl.BlockSpec((NC, BN, EDGE_DIM), lambda i: (0, i, 0)),
            full((EDGE_DIM, EDGE_DIM)),
            full((1, EDGE_DIM)),
            full((EDGE_DIM, EDGE_DIM)),
            full((1, EDGE_DIM)),
            full((EDGE_DIM, EDGE_DIM)),
            full((1, EDGE_DIM)),
            full((EDGE_DIM, 1)),
        ],
        out_specs=pl.BlockSpec((BN, 1), lambda i: (i, 0)),
        out_shape=jax.ShapeDtypeStruct((N_NODES_STATIC, 1), jnp.float32),
    )(partials, W1, b1.reshape(1, -1), W2, b2.reshape(1, -1),
      W3, b3.reshape(1, -1), W_out)


def kernel(x, rbf, idx_i, num_nodes, W_rbf, W1, b1, W2, b2, W3, b3, W_out):
    idx = idx_i.astype(jnp.int32) + (
        jnp.asarray(num_nodes, jnp.int32) - N_NODES_STATIC)
    zeros = jnp.zeros((N_NODES_STATIC, EDGE_DIM), jnp.float32)

    h = _gating(rbf, x, W_rbf)
    partials = _sc_scatter(h, idx, zeros)
    return _mlp(partials, W1, b1, W2, b2, W3, b3, W_out)


# speedup vs baseline: 3.1404x; 1.2138x over previous
import functools

import jax
import jax.numpy as jnp
from jax import lax
from jax.experimental import pallas as pl
from jax.experimental.pallas import tpu as pltpu
from jax.experimental.pallas import tpu_sc as plsc

N_NODES_STATIC = 10000
N_EDGES = 320000
EDGE_DIM = 128
N_RADIAL = 16

NC = 2
NS = 16
NW = NC * NS

GROUP = 128
N_GROUPS = N_EDGES // GROUP
RPW = N_GROUPS // NW
N_TAIL = N_GROUPS - NW * RPW
ZROWS = 624
ZTAIL = N_NODES_STATIC - NS * ZROWS


def _gate_body(rbf_ref, x_ref, w_ref, h_ref):
    g = jnp.dot(rbf_ref[...], w_ref[...], preferred_element_type=jnp.float32)
    h_ref[...] = g * x_ref[...]


def _gating(rbf, x, W_rbf):
    BE = 16000
    grid = (N_EDGES // BE,)
    return pl.pallas_call(
        _gate_body,
        grid=grid,
        in_specs=[
            pl.BlockSpec((BE, N_RADIAL), lambda i: (i, 0)),
            pl.BlockSpec((BE, EDGE_DIM), lambda i: (i, 0)),
            pl.BlockSpec((N_RADIAL, EDGE_DIM), lambda i: (0, 0)),
        ],
        out_specs=pl.BlockSpec((BE, EDGE_DIM), lambda i: (i, 0)),
        out_shape=jax.ShapeDtypeStruct((N_EDGES, EDGE_DIM), jnp.float32),
    )(rbf, x, W_rbf)


def _sc_scatter_body(h_hbm, idx_hbm, zeros_hbm, out_hbm, idx_v, rows_v, acc,
                     sem0, sem1):
    c = lax.axis_index("c")
    s = lax.axis_index("s")
    w = s * NC + c

    pltpu.sync_copy(
        zeros_hbm.at[pl.ds(s * ZROWS, ZROWS)],
        acc.at[pl.ds(s * ZROWS, ZROWS)],
    )

    @pl.when(s == NS - 1)
    def _zero_tail():
        pltpu.sync_copy(
            zeros_hbm.at[pl.ds(NS * ZROWS, ZTAIL)],
            acc.at[pl.ds(NS * ZROWS, ZTAIL)],
        )

    plsc.subcore_barrier()

    g0 = w * RPW
    sems = (sem0, sem1)

    def start(g, b):
        sem = sems[b]
        pltpu.async_copy(h_hbm.at[pl.ds(g * GROUP, GROUP)], rows_v.at[b], sem)
        pltpu.async_copy(idx_hbm.at[pl.ds(g * GROUP, GROUP)], idx_v.at[b], sem)

    def wait_and_scatter(b):
        sem = sems[b]
        pltpu.make_async_copy(
            h_hbm.at[pl.ds(0, GROUP)], rows_v.at[b], sem).wait()
        pltpu.make_async_copy(
            idx_hbm.at[pl.ds(0, GROUP)], idx_v.at[b], sem).wait()
        pltpu.sync_copy(rows_v.at[b], acc.at[idx_v.at[b]], add=True)

    start(g0, 0)

    def group_body(t, carry):
        start(g0 + 2 * t + 1, 1)
        wait_and_scatter(0)

        @pl.when(t < RPW // 2 - 1)
        def _start_next():
            start(g0 + 2 * t + 2, 0)

        wait_and_scatter(1)
        return carry

    lax.fori_loop(0, RPW // 2, group_body, 0)

    @pl.when(w < N_TAIL)
    def _tail():
        gt = NW * RPW + w
        pltpu.sync_copy(h_hbm.at[pl.ds(gt * GROUP, GROUP)], rows_v.at[0])
        pltpu.sync_copy(idx_hbm.at[pl.ds(gt * GROUP, GROUP)], idx_v.at[0])
        pltpu.sync_copy(rows_v.at[0], acc.at[idx_v.at[0]], add=True)

    plsc.subcore_barrier()
    pltpu.sync_copy(
        acc.at[pl.ds(s * ZROWS, ZROWS)],
        out_hbm.at[c, pl.ds(s * ZROWS, ZROWS)],
    )

    @pl.when(s == NS - 1)
    def _write_tail():
        pltpu.sync_copy(
            acc.at[pl.ds(NS * ZROWS, ZTAIL)],
            out_hbm.at[c, pl.ds(NS * ZROWS, ZTAIL)],
        )


def _sc_scatter(h, idx1d, zeros):
    mesh = plsc.VectorSubcoreMesh(core_axis_name="c", subcore_axis_name="s")
    kfn = pl.kernel(
        _sc_scatter_body,
        out_type=jax.ShapeDtypeStruct((NC, N_NODES_STATIC, EDGE_DIM), jnp.float32),
        mesh=mesh,
        scratch_types=[
            pltpu.VMEM((2, GROUP), jnp.int32),
            pltpu.VMEM((2, GROUP, EDGE_DIM), jnp.float32),
            pltpu.VMEM_SHARED((N_NODES_STATIC, EDGE_DIM), jnp.float32),
            pltpu.SemaphoreType.DMA,
            pltpu.SemaphoreType.DMA,
        ],
    )
    return kfn(h, idx1d, zeros)


def _mlp_body(p_ref, w1, b1, w2, b2, w3, b3, wo, o_ref):
    a = p_ref[0] + p_ref[1]
    for wref, bref in ((w1, b1), (w2, b2), (w3, b3)):
        a = jnp.dot(a, wref[...], preferred_element_type=jnp.float32) + bref[...]
        a = a * (1.0 / (1.0 + jnp.exp(-a)))
    o_ref[...] = jnp.dot(a, wo[...], preferred_element_type=jnp.float32)


def _mlp(partials, W1, b1, W2, b2, W3, b3, W_out):
    BN = 2000
    grid = (N_NODES_STATIC // BN,)
    full = lambda shape: pl.BlockSpec(shape, lambda i: tuple(0 for _ in shape))
    return pl.pallas_call(
        _mlp_body,
        grid=grid,
        in_specs=[
            pl.BlockSpec((NC, BN, EDGE_DIM), lambda i: (0, i, 0)),
            full((EDGE_DIM, EDGE_DIM)),
            full((1, EDGE_DIM)),
            full((EDGE_DIM, EDGE_DIM)),
            full((1, EDGE_DIM)),
            full((EDGE_DIM, EDGE_DIM)),
            full((1, EDGE_DIM)),
            full((EDGE_DIM, 1)),
        ],
        out_specs=pl.BlockSpec((BN, 1), lambda i: (i, 0)),
        out_shape=jax.ShapeDtypeStruct((N_NODES_STATIC, 1), jnp.float32),
    )(partials, W1, b1.reshape(1, -1), W2, b2.reshape(1, -1),
      W3, b3.reshape(1, -1), W_out)


def kernel(x, rbf, idx_i, num_nodes, W_rbf, W1, b1, W2, b2, W3, b3, W_out):
    idx = idx_i.astype(jnp.int32) + (
        jnp.asarray(num_nodes, jnp.int32) - N_NODES_STATIC)
    zeros = jnp.zeros((N_NODES_STATIC, EDGE_DIM), jnp.float32)

    h = _gating(rbf, x, W_rbf)
    partials = _sc_scatter(h, idx, zeros)
    return _mlp(partials, W1, b1, W2, b2, W3, b3, W_out)


# speedup vs baseline: 3.2123x; 1.0229x over previous
import functools

import jax
import jax.numpy as jnp
from jax import lax
from jax.experimental import pallas as pl
from jax.experimental.pallas import tpu as pltpu
from jax.experimental.pallas import tpu_sc as plsc

N_NODES_STATIC = 10000
N_EDGES = 320000
EDGE_DIM = 128
N_RADIAL = 16

NC = 2
NS = 16
NW = NC * NS

GROUP = 128
N_CHUNKS = 2
CHUNK_EDGES = N_EDGES // N_CHUNKS
CHUNK_GROUPS = CHUNK_EDGES // GROUP
RPW = CHUNK_GROUPS // NW
N_TAIL = CHUNK_GROUPS - NW * RPW
ZROWS = 624
ZTAIL = N_NODES_STATIC - NS * ZROWS


def _gate_body(rbf_ref, x_ref, w_ref, h_ref):
    g = jnp.dot(rbf_ref[...], w_ref[...], preferred_element_type=jnp.float32)
    h_ref[...] = g * x_ref[...]


def _gating(rbf, x, W_rbf, chunk):
    BE = 16000
    nb = CHUNK_EDGES // BE
    base = chunk * nb
    return pl.pallas_call(
        _gate_body,
        grid=(nb,),
        in_specs=[
            pl.BlockSpec((BE, N_RADIAL), lambda i: (base + i, 0)),
            pl.BlockSpec((BE, EDGE_DIM), lambda i: (base + i, 0)),
            pl.BlockSpec((N_RADIAL, EDGE_DIM), lambda i: (0, 0)),
        ],
        out_specs=pl.BlockSpec((BE, EDGE_DIM), lambda i: (i, 0)),
        out_shape=jax.ShapeDtypeStruct((CHUNK_EDGES, EDGE_DIM), jnp.float32),
    )(rbf, x, W_rbf)


def _sc_scatter_body(h_hbm, idx_hbm, out_hbm, idx_v, rows_v, acc, sem0, sem1):
    c = lax.axis_index("c")
    s = lax.axis_index("s")
    w = s * NC + c

    z16 = jnp.zeros((16,), jnp.float32)

    def zero_body(t, carry):
        rows_v[0, t // 8, pl.ds((t % 8) * 16, 16)] = z16
        return carry

    lax.fori_loop(0, GROUP * 8, zero_body, 0)
    for zi, zn in ((0, 128), (128, 128), (256, 128), (384, 128), (512, 112)):
        pltpu.sync_copy(
            rows_v.at[0].at[pl.ds(0, zn)],
            acc.at[pl.ds(s * ZROWS + zi, zn)],
        )

    @pl.when(s == NS - 1)
    def _zero_tail():
        pltpu.sync_copy(
            rows_v.at[0].at[pl.ds(0, ZTAIL)],
            acc.at[pl.ds(NS * ZROWS, ZTAIL)],
        )

    plsc.subcore_barrier()

    g0 = w * RPW
    sems = (sem0, sem1)

    def start(g, b):
        sem = sems[b]
        pltpu.async_copy(h_hbm.at[pl.ds(g * GROUP, GROUP)], rows_v.at[b], sem)
        pltpu.async_copy(idx_hbm.at[pl.ds(g * GROUP, GROUP)], idx_v.at[b], sem)

    def wait_and_scatter(b):
        sem = sems[b]
        pltpu.make_async_copy(
            h_hbm.at[pl.ds(0, GROUP)], rows_v.at[b], sem).wait()
        pltpu.make_async_copy(
            idx_hbm.at[pl.ds(0, GROUP)], idx_v.at[b], sem).wait()
        pltpu.sync_copy(rows_v.at[b], acc.at[idx_v.at[b]], add=True)

    start(g0, 0)

    def group_body(t, carry):
        start(g0 + 2 * t + 1, 1)
        wait_and_scatter(0)

        @pl.when(2 * t + 2 < RPW)
        def _start_next():
            start(g0 + 2 * t + 2, 0)

        wait_and_scatter(1)
        return carry

    lax.fori_loop(0, RPW // 2, group_body, 0)
    if RPW % 2 == 1:
        wait_and_scatter(0)

    @pl.when(w < N_TAIL)
    def _tail():
        gt = NW * RPW + w
        pltpu.sync_copy(h_hbm.at[pl.ds(gt * GROUP, GROUP)], rows_v.at[0])
        pltpu.sync_copy(idx_hbm.at[pl.ds(gt * GROUP, GROUP)], idx_v.at[0])
        pltpu.sync_copy(rows_v.at[0], acc.at[idx_v.at[0]], add=True)

    plsc.subcore_barrier()
    pltpu.sync_copy(
        acc.at[pl.ds(s * ZROWS, ZROWS)],
        out_hbm.at[c, pl.ds(s * ZROWS, ZROWS)],
    )

    @pl.when(s == NS - 1)
    def _write_tail():
        pltpu.sync_copy(
            acc.at[pl.ds(NS * ZROWS, ZTAIL)],
            out_hbm.at[c, pl.ds(NS * ZROWS, ZTAIL)],
        )


def _sc_scatter(h, idx1d):
    mesh = plsc.VectorSubcoreMesh(core_axis_name="c", subcore_axis_name="s")
    kfn = pl.kernel(
        _sc_scatter_body,
        out_type=jax.ShapeDtypeStruct((NC, N_NODES_STATIC, EDGE_DIM), jnp.float32),
        mesh=mesh,
        scratch_types=[
            pltpu.VMEM((2, GROUP), jnp.int32),
            pltpu.VMEM((2, GROUP, EDGE_DIM), jnp.float32),
            pltpu.VMEM_SHARED((N_NODES_STATIC, EDGE_DIM), jnp.float32),
            pltpu.SemaphoreType.DMA,
            pltpu.SemaphoreType.DMA,
        ],
    )
    return kfn(h, idx1d)


def _mlp_body(p0_ref, p1_ref, w1, b1, w2, b2, w3, b3, wo, o_ref):
    a = (p0_ref[0] + p0_ref[1]) + (p1_ref[0] + p1_ref[1])
    for wref, bref in ((w1, b1), (w2, b2), (w3, b3)):
        a = jnp.dot(a, wref[...], preferred_element_type=jnp.float32) + bref[...]
        a = a * (1.0 / (1.0 + jnp.exp(-a)))
    o_ref[...] = jnp.dot(a, wo[...], preferred_element_type=jnp.float32)


def _mlp(p0, p1, W1, b1, W2, b2, W3, b3, W_out):
    BN = 2000
    grid = (N_NODES_STATIC // BN,)
    full = lambda shape: pl.BlockSpec(shape, lambda i: tuple(0 for _ in shape))
    return pl.pallas_call(
        _mlp_body,
        grid=grid,
        in_specs=[
            pl.BlockSpec((NC, BN, EDGE_DIM), lambda i: (0, i, 0)),
            pl.BlockSpec((NC, BN, EDGE_DIM), lambda i: (0, i, 0)),
            full((EDGE_DIM, EDGE_DIM)),
            full((1, EDGE_DIM)),
            full((EDGE_DIM, EDGE_DIM)),
            full((1, EDGE_DIM)),
            full((EDGE_DIM, EDGE_DIM)),
            full((1, EDGE_DIM)),
            full((EDGE_DIM, 1)),
        ],
        out_specs=pl.BlockSpec((BN, 1), lambda i: (i, 0)),
        out_shape=jax.ShapeDtypeStruct((N_NODES_STATIC, 1), jnp.float32),
    )(p0, p1, W1, b1.reshape(1, -1), W2, b2.reshape(1, -1),
      W3, b3.reshape(1, -1), W_out)


def kernel(x, rbf, idx_i, num_nodes, W_rbf, W1, b1, W2, b2, W3, b3, W_out):
    idx = idx_i.astype(jnp.int32) + (
        jnp.asarray(num_nodes, jnp.int32) - N_NODES_STATIC)

    partials = []
    for chunk in range(N_CHUNKS):
        h_c = _gating(rbf, x, W_rbf, chunk)
        idx_c = lax.slice(idx, (chunk * CHUNK_EDGES,),
                          ((chunk + 1) * CHUNK_EDGES,))
        partials.append(_sc_scatter(h_c, idx_c))
    return _mlp(partials[0], partials[1], W1, b1, W2, b2, W3, b3, W_out)
